# baseline (speedup 1.0000x reference)
import functools

import jax
import jax.numpy as jnp
from jax.experimental import pallas as pl
from jax.experimental.pallas import tpu as pltpu

_D = 128
_B = 64
_T = 4
_BS = 4000


def _set2set_body(nblk, x_ref, batch_ref, wih_ref, whh_ref, bias_ref, out_ref):
    D, B, T, BS = _D, _B, _T, _BS
    f32 = jnp.float32
    hi = jax.lax.Precision.HIGHEST

    h = jnp.zeros((B, D), f32)
    c = jnp.zeros((B, D), f32)
    q_star = jnp.zeros((B, 2 * D), f32)
    seg_ids = jax.lax.broadcasted_iota(jnp.int32, (B, 1), 0)

    for _ in range(T):
        gates = (
            jnp.dot(q_star, wih_ref[...], precision=hi, preferred_element_type=f32)
            + jnp.dot(h, whh_ref[...], precision=hi, preferred_element_type=f32)
            + bias_ref[...]
        )
        ig = jax.nn.sigmoid(gates[:, :D])
        fg = jax.nn.sigmoid(gates[:, D : 2 * D])
        gg = jnp.tanh(gates[:, 2 * D : 3 * D])
        og = jax.nn.sigmoid(gates[:, 3 * D :])
        c = fg * c + ig * gg
        h = og * jnp.tanh(c)
        q = h

        def blk(j, carry):
            m, den, rnum = carry
            xb = x_ref[pl.ds(j * BS, BS), :]
            bb = batch_ref[j]
            ET = jax.lax.dot_general(
                q, xb, (((1,), (1,)), ((), ())), precision=hi,
                preferred_element_type=f32,
            )
            m_new = jnp.maximum(m, jnp.max(ET, keepdims=True))
            scale = jnp.exp(m - m_new)
            P = jnp.where(bb == seg_ids, jnp.exp(ET - m_new), 0.0)
            den = den * scale + jnp.sum(P, axis=1, keepdims=True)
            rnum = rnum * scale + jax.lax.dot_general(
                P, xb, (((1,), (0,)), ((), ())), precision=hi,
                preferred_element_type=f32,
            )
            return m_new, den, rnum

        m0 = jnp.full((1, 1), -1e30, f32)
        m, den, rnum = jax.lax.fori_loop(
            0, nblk, blk, (m0, jnp.zeros((B, 1), f32), jnp.zeros((B, D), f32))
        )
        r = rnum / jnp.maximum(den, 1e-30)
        q_star = jnp.concatenate([q, r], axis=1)

    out_ref[...] = q_star


def kernel(x, batch, W_ih, W_hh, b_ih, b_hh):
    n, d = x.shape
    assert d == _D and n % _BS == 0
    nblk = n // _BS
    batch3 = batch.reshape(nblk, 1, _BS)
    wihT = W_ih.T
    whhT = W_hh.T
    bias = (b_ih + b_hh).reshape(1, 4 * _D)

    return pl.pallas_call(
        functools.partial(_set2set_body, nblk),
        out_shape=jax.ShapeDtypeStruct((_B, 2 * _D), x.dtype),
        compiler_params=pltpu.CompilerParams(
            vmem_limit_bytes=64 * 1024 * 1024,
        ),
    )(x, batch3, wihT, whhT, bias)


# speedup vs baseline: 1.0597x; 1.0597x over previous
import functools

import jax
import jax.numpy as jnp
from jax.experimental import pallas as pl
from jax.experimental.pallas import tpu as pltpu

_D = 128
_B = 64
_T = 4
_BS = 5000


def _set2set_body(nblk, x_ref, batch_ref, wih_ref, whh_ref, bias_ref, out_ref):
    D, B, T, BS = _D, _B, _T, _BS
    f32 = jnp.float32
    hi = jax.lax.Precision.HIGHEST

    h = jnp.zeros((B, D), f32)
    c = jnp.zeros((B, D), f32)
    q_star = jnp.zeros((B, 2 * D), f32)
    seg_ids = jax.lax.broadcasted_iota(jnp.int32, (B, 1), 0)

    for _ in range(T):
        gates = (
            jnp.dot(q_star, wih_ref[...], precision=hi, preferred_element_type=f32)
            + jnp.dot(h, whh_ref[...], precision=hi, preferred_element_type=f32)
            + bias_ref[...]
        )
        ig = jax.nn.sigmoid(gates[:, :D])
        fg = jax.nn.sigmoid(gates[:, D : 2 * D])
        gg = jnp.tanh(gates[:, 2 * D : 3 * D])
        og = jax.nn.sigmoid(gates[:, 3 * D :])
        c = fg * c + ig * gg
        h = og * jnp.tanh(c)
        q = h

        def blk(j, carry):
            m, den, rnum = carry
            xb = x_ref[pl.ds(j * BS, BS), :]
            bb = batch_ref[j]
            mbias = jnp.where(bb == seg_ids, 0.0, -jnp.inf)
            ET = jax.lax.dot_general(
                q, xb, (((1,), (1,)), ((), ())), precision=hi,
                preferred_element_type=f32,
            )
            Em = ET + mbias
            m_new = jnp.maximum(m, jnp.max(Em, axis=1, keepdims=True))
            scale = jnp.exp(m - m_new)
            P = jnp.exp(Em - m_new)
            den = den * scale + jnp.sum(P, axis=1, keepdims=True)
            rnum = rnum * scale + jax.lax.dot_general(
                P, xb, (((1,), (0,)), ((), ())), precision=hi,
                preferred_element_type=f32,
            )
            return m_new, den, rnum

        m0 = jnp.full((B, 1), -1e30, f32)
        m, den, rnum = jax.lax.fori_loop(
            0, nblk, blk, (m0, jnp.zeros((B, 1), f32), jnp.zeros((B, D), f32))
        )
        r = rnum / jnp.maximum(den, 1e-30)
        q_star = jnp.concatenate([q, r], axis=1)

    out_ref[...] = q_star


def kernel(x, batch, W_ih, W_hh, b_ih, b_hh):
    n, d = x.shape
    assert d == _D and n % _BS == 0
    nblk = n // _BS
    batch3 = batch.reshape(nblk, 1, _BS)
    wihT = W_ih.T
    whhT = W_hh.T
    bias = (b_ih + b_hh).reshape(1, 4 * _D)

    return pl.pallas_call(
        functools.partial(_set2set_body, nblk),
        out_shape=jax.ShapeDtypeStruct((_B, 2 * _D), x.dtype),
        compiler_params=pltpu.CompilerParams(
            vmem_limit_bytes=64 * 1024 * 1024,
        ),
    )(x, batch3, wihT, whhT, bias)


# speedup vs baseline: 1.2826x; 1.2103x over previous
import functools

import jax
import jax.numpy as jnp
from jax.experimental import pallas as pl
from jax.experimental.pallas import tpu as pltpu

_D = 128
_B = 64
_T = 4
_BS = 5000


def _set2set_body(nblk, x_ref, batch_ref, wih_ref, whh_ref, bias_ref, out_ref):
    D, B, T, BS = _D, _B, _T, _BS
    f32 = jnp.float32
    hi = jax.lax.Precision.HIGHEST

    h = jnp.zeros((B, D), f32)
    c = jnp.zeros((B, D), f32)
    q_star = jnp.zeros((B, 2 * D), f32)
    seg_ids = jax.lax.broadcasted_iota(jnp.int32, (B, 1), 0)

    for _ in range(T):
        gates = (
            jnp.dot(q_star, wih_ref[...], precision=hi, preferred_element_type=f32)
            + jnp.dot(h, whh_ref[...], precision=hi, preferred_element_type=f32)
            + bias_ref[...]
        )
        ig = jax.nn.sigmoid(gates[:, :D])
        fg = jax.nn.sigmoid(gates[:, D : 2 * D])
        gg = jnp.tanh(gates[:, 2 * D : 3 * D])
        og = jax.nn.sigmoid(gates[:, 3 * D :])
        c = fg * c + ig * gg
        h = og * jnp.tanh(c)
        q = h

        def blk(j, carry):
            m, den, rnum = carry
            xb = x_ref[pl.ds(j * BS, BS), :]
            bb = batch_ref[j]
            mbias = jnp.where(bb == seg_ids, 0.0, -jnp.inf)
            ET = jax.lax.dot_general(
                q, xb, (((1,), (1,)), ((), ())), precision=hi,
                preferred_element_type=f32,
            )
            Em = ET + mbias
            m_new = jnp.maximum(m, jnp.max(Em, axis=1, keepdims=True))
            scale = jnp.exp(m - m_new)
            P = jnp.exp(Em - m_new)
            den = den * scale + jnp.sum(P, axis=1, keepdims=True)
            rnum = rnum * scale + jax.lax.dot_general(
                P, xb, (((1,), (0,)), ((), ())),
                precision=jax.lax.Precision.DEFAULT,
                preferred_element_type=f32,
            )
            return m_new, den, rnum

        m0 = jnp.full((B, 1), -1e30, f32)
        m, den, rnum = jax.lax.fori_loop(
            0, nblk, blk, (m0, jnp.zeros((B, 1), f32), jnp.zeros((B, D), f32))
        )
        r = rnum / jnp.maximum(den, 1e-30)
        q_star = jnp.concatenate([q, r], axis=1)

    out_ref[...] = q_star


def kernel(x, batch, W_ih, W_hh, b_ih, b_hh):
    n, d = x.shape
    assert d == _D and n % _BS == 0
    nblk = n // _BS
    batch3 = batch.reshape(nblk, 1, _BS)
    wihT = W_ih.T
    whhT = W_hh.T
    bias = (b_ih + b_hh).reshape(1, 4 * _D)

    return pl.pallas_call(
        functools.partial(_set2set_body, nblk),
        out_shape=jax.ShapeDtypeStruct((_B, 2 * _D), x.dtype),
        compiler_params=pltpu.CompilerParams(
            vmem_limit_bytes=64 * 1024 * 1024,
        ),
    )(x, batch3, wihT, whhT, bias)


# speedup vs baseline: 1.3251x; 1.0332x over previous
import functools

import jax
import jax.numpy as jnp
from jax.experimental import pallas as pl
from jax.experimental.pallas import tpu as pltpu

_D = 128
_B = 64
_T = 4
_BS = 5000


def _set2set_body(nblk, x_ref, batch_ref, wih_ref, whh_ref, bias_ref, out_ref):
    D, B, T, BS = _D, _B, _T, _BS
    f32 = jnp.float32
    hi = jax.lax.Precision.HIGHEST

    h = jnp.zeros((B, D), f32)
    c = jnp.zeros((B, D), f32)
    q_star = jnp.zeros((B, 2 * D), f32)
    seg_ids = jax.lax.broadcasted_iota(jnp.int32, (B, 1), 0)

    for _ in range(T):
        gates = (
            jnp.dot(q_star, wih_ref[...], precision=hi, preferred_element_type=f32)
            + jnp.dot(h, whh_ref[...], precision=hi, preferred_element_type=f32)
            + bias_ref[...]
        )
        ig = jax.nn.sigmoid(gates[:, :D])
        fg = jax.nn.sigmoid(gates[:, D : 2 * D])
        gg = jnp.tanh(gates[:, 2 * D : 3 * D])
        og = jax.nn.sigmoid(gates[:, 3 * D :])
        c = fg * c + ig * gg
        h = og * jnp.tanh(c)
        q = h

        def one_block(j, m, den, rnum):
            xb = x_ref[pl.ds(j * BS, BS), :]
            bb = batch_ref[j]
            mbias = jnp.where(bb == seg_ids, 0.0, -jnp.inf)
            ET = jax.lax.dot_general(
                q, xb, (((1,), (1,)), ((), ())), precision=hi,
                preferred_element_type=f32,
            )
            Em = ET + mbias
            m_new = jnp.maximum(m, jnp.max(Em, axis=1, keepdims=True))
            scale = jnp.exp(m - m_new)
            P = jnp.exp(Em - m_new)
            den = den * scale + jnp.sum(P, axis=1, keepdims=True)
            rnum = rnum * scale + jax.lax.dot_general(
                P, xb, (((1,), (0,)), ((), ())),
                precision=jax.lax.Precision.DEFAULT,
                preferred_element_type=f32,
            )
            return m_new, den, rnum

        half = nblk // 2

        def blk(j, carry):
            m1, den1, rnum1, m2, den2, rnum2 = carry
            m1, den1, rnum1 = one_block(j, m1, den1, rnum1)
            m2, den2, rnum2 = one_block(half + j, m2, den2, rnum2)
            return m1, den1, rnum1, m2, den2, rnum2

        m0 = jnp.full((B, 1), -1e30, f32)
        z1 = jnp.zeros((B, 1), f32)
        zD = jnp.zeros((B, D), f32)
        m1, den1, rnum1, m2, den2, rnum2 = jax.lax.fori_loop(
            0, half, blk, (m0, z1, zD, m0, z1, zD)
        )
        m = jnp.maximum(m1, m2)
        s1 = jnp.exp(m1 - m)
        s2 = jnp.exp(m2 - m)
        den = den1 * s1 + den2 * s2
        rnum = rnum1 * s1 + rnum2 * s2
        r = rnum / jnp.maximum(den, 1e-30)
        q_star = jnp.concatenate([q, r], axis=1)

    out_ref[...] = q_star


def kernel(x, batch, W_ih, W_hh, b_ih, b_hh):
    n, d = x.shape
    assert d == _D and n % _BS == 0
    nblk = n // _BS
    batch3 = batch.reshape(nblk, 1, _BS)
    wihT = W_ih.T
    whhT = W_hh.T
    bias = (b_ih + b_hh).reshape(1, 4 * _D)

    return pl.pallas_call(
        functools.partial(_set2set_body, nblk),
        out_shape=jax.ShapeDtypeStruct((_B, 2 * _D), x.dtype),
        compiler_params=pltpu.CompilerParams(
            vmem_limit_bytes=64 * 1024 * 1024,
        ),
    )(x, batch3, wihT, whhT, bias)


# speedup vs baseline: 2.3374x; 1.7639x over previous
import functools

import jax
import jax.numpy as jnp
from jax.experimental import pallas as pl
from jax.experimental.pallas import tpu as pltpu

_D = 128
_B = 64
_T = 4
_BS = 5000


def _set2set_body(nblk, x_ref, batch_ref, wih_ref, whh_ref, bias_ref, out_ref):
    D, B, T, BS = _D, _B, _T, _BS
    f32 = jnp.float32
    hi = jax.lax.Precision.HIGHEST

    h = jnp.zeros((B, D), f32)
    c = jnp.zeros((B, D), f32)
    q_star = jnp.zeros((B, 2 * D), f32)
    seg_ids = jax.lax.broadcasted_iota(jnp.int32, (B, 1), 0)

    for _ in range(T):
        gates = (
            jnp.dot(q_star, wih_ref[...], precision=hi, preferred_element_type=f32)
            + jnp.dot(h, whh_ref[...], precision=hi, preferred_element_type=f32)
            + bias_ref[...]
        )
        ig = jax.nn.sigmoid(gates[:, :D])
        fg = jax.nn.sigmoid(gates[:, D : 2 * D])
        gg = jnp.tanh(gates[:, 2 * D : 3 * D])
        og = jax.nn.sigmoid(gates[:, 3 * D :])
        c = fg * c + ig * gg
        h = og * jnp.tanh(c)
        q = h
        q_hi = q.astype(jnp.bfloat16)
        q_lo = (q - q_hi.astype(f32)).astype(jnp.bfloat16)

        def one_block(j, m, den, rnum):
            xb = x_ref[pl.ds(j * BS, BS), :]
            bb = batch_ref[j]
            mbias = jnp.where(bb == seg_ids, 0.0, -jnp.inf)
            dn = (((1,), (1,)), ((), ()))
            ET = jax.lax.dot_general(
                q_hi, xb, dn, precision=jax.lax.Precision.DEFAULT,
                preferred_element_type=f32,
            ) + jax.lax.dot_general(
                q_lo, xb, dn, precision=jax.lax.Precision.DEFAULT,
                preferred_element_type=f32,
            )
            Em = ET + mbias
            m_new = jnp.maximum(m, jnp.max(Em, axis=1, keepdims=True))
            scale = jnp.exp(m - m_new)
            P = jnp.exp(Em - m_new)
            den = den * scale + jnp.sum(P, axis=1, keepdims=True)
            rnum = rnum * scale + jax.lax.dot_general(
                P.astype(jnp.bfloat16), xb, (((1,), (0,)), ((), ())),
                precision=jax.lax.Precision.DEFAULT,
                preferred_element_type=f32,
            )
            return m_new, den, rnum

        half = nblk // 2

        def blk(j, carry):
            m1, den1, rnum1, m2, den2, rnum2 = carry
            m1, den1, rnum1 = one_block(j, m1, den1, rnum1)
            m2, den2, rnum2 = one_block(half + j, m2, den2, rnum2)
            return m1, den1, rnum1, m2, den2, rnum2

        m0 = jnp.full((B, 1), -1e30, f32)
        z1 = jnp.zeros((B, 1), f32)
        zD = jnp.zeros((B, D), f32)
        m1, den1, rnum1, m2, den2, rnum2 = jax.lax.fori_loop(
            0, half, blk, (m0, z1, zD, m0, z1, zD)
        )
        m = jnp.maximum(m1, m2)
        s1 = jnp.exp(m1 - m)
        s2 = jnp.exp(m2 - m)
        den = den1 * s1 + den2 * s2
        rnum = rnum1 * s1 + rnum2 * s2
        r = rnum / jnp.maximum(den, 1e-30)
        q_star = jnp.concatenate([q, r], axis=1)

    out_ref[...] = q_star


def kernel(x, batch, W_ih, W_hh, b_ih, b_hh):
    n, d = x.shape
    assert d == _D and n % _BS == 0
    nblk = n // _BS
    x_hi = x.astype(jnp.bfloat16)
    batch3 = batch.reshape(nblk, 1, _BS)
    wihT = W_ih.T
    whhT = W_hh.T
    bias = (b_ih + b_hh).reshape(1, 4 * _D)

    return pl.pallas_call(
        functools.partial(_set2set_body, nblk),
        out_shape=jax.ShapeDtypeStruct((_B, 2 * _D), x.dtype),
        compiler_params=pltpu.CompilerParams(
            vmem_limit_bytes=64 * 1024 * 1024,
        ),
    )(x_hi, batch3, wihT, whhT, bias)


# speedup vs baseline: 2.5849x; 1.1059x over previous
import functools

import jax
import jax.numpy as jnp
from jax.experimental import pallas as pl
from jax.experimental.pallas import tpu as pltpu

_D = 128
_B = 64
_T = 4
_BS = 10000


def _set2set_body(nblk, x_ref, batch_ref, wih_ref, whh_ref, bias_ref, out_ref):
    D, B, T, BS = _D, _B, _T, _BS
    f32 = jnp.float32
    hi = jax.lax.Precision.HIGHEST

    h = jnp.zeros((B, D), f32)
    c = jnp.zeros((B, D), f32)
    q_star = jnp.zeros((B, 2 * D), f32)
    seg_ids = jax.lax.broadcasted_iota(jnp.int32, (B, 1), 0)

    for _ in range(T):
        gates = (
            jnp.dot(q_star, wih_ref[...], precision=hi, preferred_element_type=f32)
            + jnp.dot(h, whh_ref[...], precision=hi, preferred_element_type=f32)
            + bias_ref[...]
        )
        ig = jax.nn.sigmoid(gates[:, :D])
        fg = jax.nn.sigmoid(gates[:, D : 2 * D])
        gg = jnp.tanh(gates[:, 2 * D : 3 * D])
        og = jax.nn.sigmoid(gates[:, 3 * D :])
        c = fg * c + ig * gg
        h = og * jnp.tanh(c)
        q = h
        q_hi = q.astype(jnp.bfloat16)
        q_lo = (q - q_hi.astype(f32)).astype(jnp.bfloat16)

        def one_block(j, m, den, rnum):
            xb = x_ref[pl.ds(j * BS, BS), :]
            bb = batch_ref[j]
            mbias = jnp.where(bb == seg_ids, 0.0, -jnp.inf)
            dn = (((1,), (1,)), ((), ()))
            ET = jax.lax.dot_general(
                q_hi, xb, dn, precision=jax.lax.Precision.DEFAULT,
                preferred_element_type=f32,
            ) + jax.lax.dot_general(
                q_lo, xb, dn, precision=jax.lax.Precision.DEFAULT,
                preferred_element_type=f32,
            )
            Em = ET + mbias
            m_new = jnp.maximum(m, jnp.max(Em, axis=1, keepdims=True))
            scale = jnp.exp(m - m_new)
            P = jnp.exp(Em - m_new)
            den = den * scale + jnp.sum(P, axis=1, keepdims=True)
            rnum = rnum * scale + jax.lax.dot_general(
                P.astype(jnp.bfloat16), xb, (((1,), (0,)), ((), ())),
                precision=jax.lax.Precision.DEFAULT,
                preferred_element_type=f32,
            )
            return m_new, den, rnum

        half = nblk // 2

        def blk(j, carry):
            m1, den1, rnum1, m2, den2, rnum2 = carry
            m1, den1, rnum1 = one_block(j, m1, den1, rnum1)
            m2, den2, rnum2 = one_block(half + j, m2, den2, rnum2)
            return m1, den1, rnum1, m2, den2, rnum2

        m0 = jnp.full((B, 1), -1e30, f32)
        z1 = jnp.zeros((B, 1), f32)
        zD = jnp.zeros((B, D), f32)
        m1, den1, rnum1, m2, den2, rnum2 = jax.lax.fori_loop(
            0, half, blk, (m0, z1, zD, m0, z1, zD)
        )
        m = jnp.maximum(m1, m2)
        s1 = jnp.exp(m1 - m)
        s2 = jnp.exp(m2 - m)
        den = den1 * s1 + den2 * s2
        rnum = rnum1 * s1 + rnum2 * s2
        r = rnum / jnp.maximum(den, 1e-30)
        q_star = jnp.concatenate([q, r], axis=1)

    out_ref[...] = q_star


def kernel(x, batch, W_ih, W_hh, b_ih, b_hh):
    n, d = x.shape
    assert d == _D and n % _BS == 0
    nblk = n // _BS
    x_hi = x.astype(jnp.bfloat16)
    batch3 = batch.reshape(nblk, 1, _BS)
    wihT = W_ih.T
    whhT = W_hh.T
    bias = (b_ih + b_hh).reshape(1, 4 * _D)

    return pl.pallas_call(
        functools.partial(_set2set_body, nblk),
        out_shape=jax.ShapeDtypeStruct((_B, 2 * _D), x.dtype),
        compiler_params=pltpu.CompilerParams(
            vmem_limit_bytes=64 * 1024 * 1024,
        ),
    )(x_hi, batch3, wihT, whhT, bias)


# speedup vs baseline: 2.6908x; 1.0410x over previous
import functools

import jax
import jax.numpy as jnp
from jax.experimental import pallas as pl
from jax.experimental.pallas import tpu as pltpu

_D = 128
_B = 64
_T = 4
_BS = 25000


def _set2set_body(nblk, x_ref, batch_ref, wih_ref, whh_ref, bias_ref, out_ref):
    D, B, T, BS = _D, _B, _T, _BS
    f32 = jnp.float32
    hi = jax.lax.Precision.HIGHEST

    h = jnp.zeros((B, D), f32)
    c = jnp.zeros((B, D), f32)
    q_star = jnp.zeros((B, 2 * D), f32)
    seg_ids = jax.lax.broadcasted_iota(jnp.int32, (B, 1), 0)

    for _ in range(T):
        gates = (
            jnp.dot(q_star, wih_ref[...], precision=hi, preferred_element_type=f32)
            + jnp.dot(h, whh_ref[...], precision=hi, preferred_element_type=f32)
            + bias_ref[...]
        )
        ig = jax.nn.sigmoid(gates[:, :D])
        fg = jax.nn.sigmoid(gates[:, D : 2 * D])
        gg = jnp.tanh(gates[:, 2 * D : 3 * D])
        og = jax.nn.sigmoid(gates[:, 3 * D :])
        c = fg * c + ig * gg
        h = og * jnp.tanh(c)
        q = h
        q_hi = q.astype(jnp.bfloat16)
        q_lo = (q - q_hi.astype(f32)).astype(jnp.bfloat16)

        def one_block(j, m, den, rnum):
            xb = x_ref[pl.ds(j * BS, BS), :]
            bb = batch_ref[j]
            mbias = jnp.where(bb == seg_ids, 0.0, -jnp.inf)
            dn = (((1,), (1,)), ((), ()))
            ET = jax.lax.dot_general(
                q_hi, xb, dn, precision=jax.lax.Precision.DEFAULT,
                preferred_element_type=f32,
            ) + jax.lax.dot_general(
                q_lo, xb, dn, precision=jax.lax.Precision.DEFAULT,
                preferred_element_type=f32,
            )
            Em = ET + mbias
            m_new = jnp.maximum(m, jnp.max(Em, axis=1, keepdims=True))
            scale = jnp.exp(m - m_new)
            P = jnp.exp(Em - m_new)
            den = den * scale + jnp.sum(P, axis=1, keepdims=True)
            rnum = rnum * scale + jax.lax.dot_general(
                P.astype(jnp.bfloat16), xb, (((1,), (0,)), ((), ())),
                precision=jax.lax.Precision.DEFAULT,
                preferred_element_type=f32,
            )
            return m_new, den, rnum

        half = nblk // 2

        def blk(j, carry):
            m1, den1, rnum1, m2, den2, rnum2 = carry
            m1, den1, rnum1 = one_block(j, m1, den1, rnum1)
            m2, den2, rnum2 = one_block(half + j, m2, den2, rnum2)
            return m1, den1, rnum1, m2, den2, rnum2

        m0 = jnp.full((B, 1), -1e30, f32)
        z1 = jnp.zeros((B, 1), f32)
        zD = jnp.zeros((B, D), f32)
        m1, den1, rnum1, m2, den2, rnum2 = jax.lax.fori_loop(
            0, half, blk, (m0, z1, zD, m0, z1, zD)
        )
        m = jnp.maximum(m1, m2)
        s1 = jnp.exp(m1 - m)
        s2 = jnp.exp(m2 - m)
        den = den1 * s1 + den2 * s2
        rnum = rnum1 * s1 + rnum2 * s2
        r = rnum / jnp.maximum(den, 1e-30)
        q_star = jnp.concatenate([q, r], axis=1)

    out_ref[...] = q_star


def kernel(x, batch, W_ih, W_hh, b_ih, b_hh):
    n, d = x.shape
    assert d == _D and n % _BS == 0
    nblk = n // _BS
    x_hi = x.astype(jnp.bfloat16)
    batch3 = batch.reshape(nblk, 1, _BS)
    wihT = W_ih.T
    whhT = W_hh.T
    bias = (b_ih + b_hh).reshape(1, 4 * _D)

    return pl.pallas_call(
        functools.partial(_set2set_body, nblk),
        out_shape=jax.ShapeDtypeStruct((_B, 2 * _D), x.dtype),
        compiler_params=pltpu.CompilerParams(
            vmem_limit_bytes=64 * 1024 * 1024,
        ),
    )(x_hi, batch3, wihT, whhT, bias)
